# baseline (speedup 1.0000x reference)
import jax
import jax.numpy as jnp
from jax.experimental import pallas as pl
from jax.experimental.pallas import tpu as pltpu

_BM = 200


def _support_kernel(x_ref, w1f_ref, w1s_ref, sf_ref, ss_ref):
    x = x_ref[...]
    sf_ref[...] = jnp.dot(x, w1f_ref[...], preferred_element_type=jnp.float32)
    ss_ref[...] = jnp.dot(x, w1s_ref[...], preferred_element_type=jnp.float32)


def _layer1_kernel(fadj_ref, sadj_ref, sf_ref, ss_ref, b1f_ref, b1s_ref,
                   w2f_ref, w2s_ref, wm_ref, uf_ref, us_ref):
    hf = jnp.maximum(
        jnp.dot(fadj_ref[...].astype(jnp.bfloat16),
                sf_ref[...].astype(jnp.bfloat16),
                preferred_element_type=jnp.float32)
        + b1f_ref[...], 0.0)
    hs = jnp.maximum(
        jnp.dot(sadj_ref[...].astype(jnp.bfloat16),
                ss_ref[...].astype(jnp.bfloat16),
                preferred_element_type=jnp.float32)
        + b1s_ref[...], 0.0)
    tf = jnp.dot(hf, w2f_ref[...], preferred_element_type=jnp.float32)
    ts = jnp.dot(hs, w2s_ref[...], preferred_element_type=jnp.float32)
    uf_ref[...] = jnp.dot(tf, wm_ref[32:64, :], preferred_element_type=jnp.float32)
    us_ref[...] = jnp.dot(ts, wm_ref[64:96, :], preferred_element_type=jnp.float32)


def _layer2_kernel(fadj_ref, sadj_ref, uf_ref, us_ref, z_ref, wm_ref,
                   b2f_ref, b2s_ref, bm_ref, out_ref):
    acc = jnp.dot(fadj_ref[...].astype(jnp.bfloat16),
                  uf_ref[...].astype(jnp.bfloat16),
                  preferred_element_type=jnp.float32)
    acc = acc + jnp.dot(sadj_ref[...].astype(jnp.bfloat16),
                        us_ref[...].astype(jnp.bfloat16),
                        preferred_element_type=jnp.float32)
    acc = acc + jnp.dot(z_ref[...], wm_ref[0:32, :], preferred_element_type=jnp.float32)
    const = jnp.dot(b2f_ref[...], wm_ref[32:64, :], preferred_element_type=jnp.float32)
    const = const + jnp.dot(b2s_ref[...], wm_ref[64:96, :], preferred_element_type=jnp.float32)
    out_ref[...] = acc + const + bm_ref[...]


def kernel(x, sadj, fadj, z, W1f, b1f, W2f, b2f, W1s, b1s, W2s, b2s, Wm, bm):
    n = sadj.shape[0]
    nfeat = x.shape[1]
    nhid1 = W1f.shape[1]
    nhid2 = W2f.shape[1]
    nclass = Wm.shape[1]
    nb = n // _BM

    b1f2 = b1f.reshape(1, nhid1)
    b1s2 = b1s.reshape(1, nhid1)
    b2f2 = b2f.reshape(1, nhid2)
    b2s2 = b2s.reshape(1, nhid2)
    bm2 = bm.reshape(1, nclass)

    sf, ss = pl.pallas_call(
        _support_kernel,
        out_shape=[jax.ShapeDtypeStruct((n, nhid1), jnp.float32)] * 2,
    )(x, W1f, W1s)

    uf, us = pl.pallas_call(
        _layer1_kernel,
        grid=(nb,),
        in_specs=[
            pl.BlockSpec((_BM, n), lambda i: (i, 0)),
            pl.BlockSpec((_BM, n), lambda i: (i, 0)),
            pl.BlockSpec((n, nhid1), lambda i: (0, 0)),
            pl.BlockSpec((n, nhid1), lambda i: (0, 0)),
            pl.BlockSpec((1, nhid1), lambda i: (0, 0)),
            pl.BlockSpec((1, nhid1), lambda i: (0, 0)),
            pl.BlockSpec((nhid1, nhid2), lambda i: (0, 0)),
            pl.BlockSpec((nhid1, nhid2), lambda i: (0, 0)),
            pl.BlockSpec((3 * nhid2, nclass), lambda i: (0, 0)),
        ],
        out_specs=[pl.BlockSpec((_BM, nclass), lambda i: (i, 0))] * 2,
        out_shape=[jax.ShapeDtypeStruct((n, nclass), jnp.float32)] * 2,
        compiler_params=pltpu.CompilerParams(
            dimension_semantics=("parallel",)),
    )(fadj, sadj, sf, ss, b1f2, b1s2, W2f, W2s, Wm)

    out = pl.pallas_call(
        _layer2_kernel,
        grid=(nb,),
        in_specs=[
            pl.BlockSpec((_BM, n), lambda i: (i, 0)),
            pl.BlockSpec((_BM, n), lambda i: (i, 0)),
            pl.BlockSpec((n, nclass), lambda i: (0, 0)),
            pl.BlockSpec((n, nclass), lambda i: (0, 0)),
            pl.BlockSpec((_BM, nhid2), lambda i: (i, 0)),
            pl.BlockSpec((3 * nhid2, nclass), lambda i: (0, 0)),
            pl.BlockSpec((1, nhid2), lambda i: (0, 0)),
            pl.BlockSpec((1, nhid2), lambda i: (0, 0)),
            pl.BlockSpec((1, nclass), lambda i: (0, 0)),
        ],
        out_specs=pl.BlockSpec((_BM, nclass), lambda i: (i, 0)),
        out_shape=jax.ShapeDtypeStruct((n, nclass), jnp.float32),
        compiler_params=pltpu.CompilerParams(
            dimension_semantics=("parallel",)),
    )(fadj, sadj, uf, us, z, Wm, b2f2, b2s2, bm2)

    return (out, None, None, None, None, None, None)


# speedup vs baseline: 1.0242x; 1.0242x over previous
import jax
import jax.numpy as jnp
from jax.experimental import pallas as pl
from jax.experimental.pallas import tpu as pltpu

_BM = 200


def _mgcn_kernel(x_ref, fadj_ref, sadj_ref, z_ref, w1f_ref, w1s_ref,
                 b1f_ref, b1s_ref, w2f_ref, w2s_ref, wm_ref,
                 b2f_ref, b2s_ref, bm_ref, out_ref,
                 sf_ref, ss_ref, uf_ref, us_ref):
    p = pl.program_id(0)
    i = pl.program_id(1)
    bm_rows = out_ref.shape[0]

    @pl.when(jnp.logical_and(p == 0, i == 0))
    def _():
        xv = x_ref[...]
        sf_ref[...] = jnp.dot(xv, w1f_ref[...], preferred_element_type=jnp.float32)
        ss_ref[...] = jnp.dot(xv, w1s_ref[...], preferred_element_type=jnp.float32)

    @pl.when(p == 0)
    def _():
        hf = jnp.maximum(
            jnp.dot(fadj_ref[...].astype(jnp.bfloat16),
                    sf_ref[...].astype(jnp.bfloat16),
                    preferred_element_type=jnp.float32) + b1f_ref[...], 0.0)
        hs = jnp.maximum(
            jnp.dot(sadj_ref[...].astype(jnp.bfloat16),
                    ss_ref[...].astype(jnp.bfloat16),
                    preferred_element_type=jnp.float32) + b1s_ref[...], 0.0)
        tf = jnp.dot(hf, w2f_ref[...], preferred_element_type=jnp.float32)
        ts = jnp.dot(hs, w2s_ref[...], preferred_element_type=jnp.float32)
        uf_ref[pl.ds(i * bm_rows, bm_rows), :] = jnp.dot(
            tf, wm_ref[32:64, :], preferred_element_type=jnp.float32)
        us_ref[pl.ds(i * bm_rows, bm_rows), :] = jnp.dot(
            ts, wm_ref[64:96, :], preferred_element_type=jnp.float32)

    @pl.when(p == 1)
    def _():
        acc = jnp.dot(fadj_ref[...].astype(jnp.bfloat16),
                      uf_ref[...].astype(jnp.bfloat16),
                      preferred_element_type=jnp.float32)
        acc = acc + jnp.dot(sadj_ref[...].astype(jnp.bfloat16),
                            us_ref[...].astype(jnp.bfloat16),
                            preferred_element_type=jnp.float32)
        acc = acc + jnp.dot(z_ref[...], wm_ref[0:32, :],
                            preferred_element_type=jnp.float32)
        const = jnp.dot(b2f_ref[...], wm_ref[32:64, :],
                        preferred_element_type=jnp.float32)
        const = const + jnp.dot(b2s_ref[...], wm_ref[64:96, :],
                                preferred_element_type=jnp.float32)
        out_ref[...] = acc + const + bm_ref[...]


def kernel(x, sadj, fadj, z, W1f, b1f, W2f, b2f, W1s, b1s, W2s, b2s, Wm, bm):
    n = sadj.shape[0]
    nfeat = x.shape[1]
    nhid1 = W1f.shape[1]
    nhid2 = W2f.shape[1]
    nclass = Wm.shape[1]
    nb = n // _BM

    b1f2 = b1f.reshape(1, nhid1)
    b1s2 = b1s.reshape(1, nhid1)
    b2f2 = b2f.reshape(1, nhid2)
    b2s2 = b2s.reshape(1, nhid2)
    bm2 = bm.reshape(1, nclass)

    const_spec = lambda shape: pl.BlockSpec(shape, lambda p, i: (0, 0))
    row_spec = lambda shape: pl.BlockSpec(shape, lambda p, i: (i, 0))

    out = pl.pallas_call(
        _mgcn_kernel,
        grid=(2, nb),
        in_specs=[
            const_spec((n, nfeat)),
            row_spec((_BM, n)),
            row_spec((_BM, n)),
            row_spec((_BM, nhid2)),
            const_spec((nfeat, nhid1)),
            const_spec((nfeat, nhid1)),
            const_spec((1, nhid1)),
            const_spec((1, nhid1)),
            const_spec((nhid1, nhid2)),
            const_spec((nhid1, nhid2)),
            const_spec((3 * nhid2, nclass)),
            const_spec((1, nhid2)),
            const_spec((1, nhid2)),
            const_spec((1, nclass)),
        ],
        out_specs=row_spec((_BM, nclass)),
        out_shape=jax.ShapeDtypeStruct((n, nclass), jnp.float32),
        scratch_shapes=[
            pltpu.VMEM((n, nhid1), jnp.float32),
            pltpu.VMEM((n, nhid1), jnp.float32),
            pltpu.VMEM((n, nclass), jnp.float32),
            pltpu.VMEM((n, nclass), jnp.float32),
        ],
        compiler_params=pltpu.CompilerParams(
            dimension_semantics=("arbitrary", "arbitrary")),
    )(x, fadj, sadj, z, W1f, W1s, b1f2, b1s2, W2f, W2s, Wm, b2f2, b2s2, bm2)

    return (out, None, None, None, None, None, None)


# speedup vs baseline: 1.0255x; 1.0013x over previous
import jax
import jax.numpy as jnp
from jax.experimental import pallas as pl
from jax.experimental.pallas import tpu as pltpu

_BM = 200


def _mgcn_kernel(x_ref, fadj_ref, sadj_ref, z_ref, w1f_ref, w1s_ref,
                 b1f_ref, b1s_ref, w2f_ref, w2s_ref, wm_ref,
                 b2f_ref, b2s_ref, bm_ref, out_ref,
                 sf_ref, ss_ref, uf_ref, us_ref):
    p = pl.program_id(0)
    i = pl.program_id(1)
    bm_rows = out_ref.shape[0]

    @pl.when(jnp.logical_and(p == 0, i == 0))
    def _():
        xv = x_ref[...]
        sf_ref[...] = jnp.dot(xv, w1f_ref[...], preferred_element_type=jnp.float32)
        ss_ref[...] = jnp.dot(xv, w1s_ref[...], preferred_element_type=jnp.float32)

    @pl.when(p == 0)
    def _():
        hf = jnp.maximum(
            jnp.dot(fadj_ref[...].astype(jnp.bfloat16),
                    sf_ref[...].astype(jnp.bfloat16),
                    preferred_element_type=jnp.float32) + b1f_ref[...], 0.0)
        hs = jnp.maximum(
            jnp.dot(sadj_ref[...].astype(jnp.bfloat16),
                    ss_ref[...].astype(jnp.bfloat16),
                    preferred_element_type=jnp.float32) + b1s_ref[...], 0.0)
        tf = jnp.dot(hf, w2f_ref[...], preferred_element_type=jnp.float32)
        ts = jnp.dot(hs, w2s_ref[...], preferred_element_type=jnp.float32)
        uf_ref[pl.ds(i * bm_rows, bm_rows), :] = jnp.dot(
            tf, wm_ref[32:64, :],
            preferred_element_type=jnp.float32).astype(jnp.bfloat16)
        us_ref[pl.ds(i * bm_rows, bm_rows), :] = jnp.dot(
            ts, wm_ref[64:96, :],
            preferred_element_type=jnp.float32).astype(jnp.bfloat16)

    @pl.when(p == 1)
    def _():
        acc = jnp.dot(fadj_ref[...].astype(jnp.bfloat16), uf_ref[...],
                      preferred_element_type=jnp.float32)
        acc = acc + jnp.dot(sadj_ref[...].astype(jnp.bfloat16), us_ref[...],
                            preferred_element_type=jnp.float32)
        acc = acc + jnp.dot(z_ref[...], wm_ref[0:32, :],
                            preferred_element_type=jnp.float32)
        const = jnp.dot(b2f_ref[...], wm_ref[32:64, :],
                        preferred_element_type=jnp.float32)
        const = const + jnp.dot(b2s_ref[...], wm_ref[64:96, :],
                                preferred_element_type=jnp.float32)
        out_ref[...] = acc + const + bm_ref[...]


def kernel(x, sadj, fadj, z, W1f, b1f, W2f, b2f, W1s, b1s, W2s, b2s, Wm, bm):
    n = sadj.shape[0]
    nfeat = x.shape[1]
    nhid1 = W1f.shape[1]
    nhid2 = W2f.shape[1]
    nclass = Wm.shape[1]
    nb = n // _BM

    b1f2 = b1f.reshape(1, nhid1)
    b1s2 = b1s.reshape(1, nhid1)
    b2f2 = b2f.reshape(1, nhid2)
    b2s2 = b2s.reshape(1, nhid2)
    bm2 = bm.reshape(1, nclass)

    const_spec = lambda shape: pl.BlockSpec(shape, lambda p, i: (0, 0))
    row_spec = lambda shape: pl.BlockSpec(shape, lambda p, i: (i, 0))

    out = pl.pallas_call(
        _mgcn_kernel,
        grid=(2, nb),
        in_specs=[
            const_spec((n, nfeat)),
            row_spec((_BM, n)),
            row_spec((_BM, n)),
            row_spec((_BM, nhid2)),
            const_spec((nfeat, nhid1)),
            const_spec((nfeat, nhid1)),
            const_spec((1, nhid1)),
            const_spec((1, nhid1)),
            const_spec((nhid1, nhid2)),
            const_spec((nhid1, nhid2)),
            const_spec((3 * nhid2, nclass)),
            const_spec((1, nhid2)),
            const_spec((1, nhid2)),
            const_spec((1, nclass)),
        ],
        out_specs=row_spec((_BM, nclass)),
        out_shape=jax.ShapeDtypeStruct((n, nclass), jnp.float32),
        scratch_shapes=[
            pltpu.VMEM((n, nhid1), jnp.float32),
            pltpu.VMEM((n, nhid1), jnp.float32),
            pltpu.VMEM((n, nclass), jnp.bfloat16),
            pltpu.VMEM((n, nclass), jnp.bfloat16),
        ],
        compiler_params=pltpu.CompilerParams(
            dimension_semantics=("arbitrary", "arbitrary")),
    )(x, fadj, sadj, z, W1f, W1s, b1f2, b1s2, W2f, W2s, Wm, b2f2, b2s2, bm2)

    return (out, None, None, None, None, None, None)


# speedup vs baseline: 1.0305x; 1.0049x over previous
import jax
import jax.numpy as jnp
from jax.experimental import pallas as pl
from jax.experimental.pallas import tpu as pltpu

_BM = 200


def _mgcn_kernel(x_ref, fadj_ref, sadj_ref, z_ref, w1f_ref, w1s_ref,
                 b1f_ref, b1s_ref, w2f_ref, w2s_ref, wm_ref,
                 b2f_ref, b2s_ref, bm_ref, out_ref,
                 sf_ref, ss_ref, uf_ref, us_ref):
    p = pl.program_id(0)
    i = pl.program_id(1)
    bm_rows = out_ref.shape[0]

    @pl.when(jnp.logical_and(p == 0, i == 0))
    def _():
        xv = x_ref[...]
        sf_ref[...] = jnp.dot(
            xv, w1f_ref[...],
            preferred_element_type=jnp.float32).astype(jnp.bfloat16)
        ss_ref[...] = jnp.dot(
            xv, w1s_ref[...],
            preferred_element_type=jnp.float32).astype(jnp.bfloat16)

    @pl.when(p == 0)
    def _():
        hf = jnp.maximum(
            jnp.dot(fadj_ref[...].astype(jnp.bfloat16), sf_ref[...],
                    preferred_element_type=jnp.float32) + b1f_ref[...], 0.0)
        hs = jnp.maximum(
            jnp.dot(sadj_ref[...].astype(jnp.bfloat16), ss_ref[...],
                    preferred_element_type=jnp.float32) + b1s_ref[...], 0.0)
        tf = jnp.dot(hf, w2f_ref[...], preferred_element_type=jnp.float32)
        ts = jnp.dot(hs, w2s_ref[...], preferred_element_type=jnp.float32)
        uf_ref[pl.ds(i * bm_rows, bm_rows), :] = jnp.dot(
            tf, wm_ref[32:64, :],
            preferred_element_type=jnp.float32).astype(jnp.bfloat16)
        us_ref[pl.ds(i * bm_rows, bm_rows), :] = jnp.dot(
            ts, wm_ref[64:96, :],
            preferred_element_type=jnp.float32).astype(jnp.bfloat16)

    @pl.when(p == 1)
    def _():
        acc = jnp.dot(fadj_ref[...].astype(jnp.bfloat16), uf_ref[...],
                      preferred_element_type=jnp.float32)
        acc = acc + jnp.dot(sadj_ref[...].astype(jnp.bfloat16), us_ref[...],
                            preferred_element_type=jnp.float32)
        acc = acc + jnp.dot(z_ref[...], wm_ref[0:32, :],
                            preferred_element_type=jnp.float32)
        const = jnp.dot(b2f_ref[...], wm_ref[32:64, :],
                        preferred_element_type=jnp.float32)
        const = const + jnp.dot(b2s_ref[...], wm_ref[64:96, :],
                                preferred_element_type=jnp.float32)
        out_ref[...] = acc + const + bm_ref[...]


def kernel(x, sadj, fadj, z, W1f, b1f, W2f, b2f, W1s, b1s, W2s, b2s, Wm, bm):
    n = sadj.shape[0]
    nfeat = x.shape[1]
    nhid1 = W1f.shape[1]
    nhid2 = W2f.shape[1]
    nclass = Wm.shape[1]
    nb = n // _BM

    b1f2 = b1f.reshape(1, nhid1)
    b1s2 = b1s.reshape(1, nhid1)
    b2f2 = b2f.reshape(1, nhid2)
    b2s2 = b2s.reshape(1, nhid2)
    bm2 = bm.reshape(1, nclass)

    const_spec = lambda shape: pl.BlockSpec(shape, lambda p, i: (0, 0))
    row_spec = lambda shape: pl.BlockSpec(shape, lambda p, i: (i, 0))
    p1_row_spec = lambda shape: pl.BlockSpec(shape, lambda p, i: (i * p, 0))

    out = pl.pallas_call(
        _mgcn_kernel,
        grid=(2, nb),
        in_specs=[
            const_spec((n, nfeat)),
            row_spec((_BM, n)),
            row_spec((_BM, n)),
            p1_row_spec((_BM, nhid2)),
            const_spec((nfeat, nhid1)),
            const_spec((nfeat, nhid1)),
            const_spec((1, nhid1)),
            const_spec((1, nhid1)),
            const_spec((nhid1, nhid2)),
            const_spec((nhid1, nhid2)),
            const_spec((3 * nhid2, nclass)),
            const_spec((1, nhid2)),
            const_spec((1, nhid2)),
            const_spec((1, nclass)),
        ],
        out_specs=p1_row_spec((_BM, nclass)),
        out_shape=jax.ShapeDtypeStruct((n, nclass), jnp.float32),
        scratch_shapes=[
            pltpu.VMEM((n, nhid1), jnp.bfloat16),
            pltpu.VMEM((n, nhid1), jnp.bfloat16),
            pltpu.VMEM((n, nclass), jnp.bfloat16),
            pltpu.VMEM((n, nclass), jnp.bfloat16),
        ],
        compiler_params=pltpu.CompilerParams(
            dimension_semantics=("arbitrary", "arbitrary")),
    )(x, fadj, sadj, z, W1f, W1s, b1f2, b1s2, W2f, W2s, Wm, b2f2, b2s2, bm2)

    return (out, None, None, None, None, None, None)
